# Initial kernel scaffold; baseline (speedup 1.0000x reference)
import jax
import jax.numpy as jnp
from jax.experimental import pallas as pl


def kernel(node_features, mobility_matrix, W_in, b_in, W_out, b_out, gamma, beta):
    raise NotImplementedError("write your pallas kernel here")


# speedup vs baseline: 1.7093x; 1.7093x over previous
import functools

import jax
import jax.numpy as jnp
from jax.experimental import pallas as pl
from jax.experimental.pallas import tpu as pltpu

_EPS = 1e-8
_THRESHOLD = 1e-6
_LN_EPS = 1e-5


def _prep_kernel(m_ref, x_ref, w_in_ref, b_in_ref, c_ref, t_ref):
    j = pl.program_id(0)

    @pl.when(j == 0)
    def _():
        c_ref[...] = jnp.zeros_like(c_ref)

    c_ref[...] += jnp.sum(m_ref[...], axis=0, keepdims=True)
    t_ref[...] = (
        jax.lax.dot_general(
            x_ref[...], w_in_ref[...], (((1,), (1,)), ((), ())),
            preferred_element_type=jnp.float32,
        )
        + b_in_ref[...]
    )


def _main_kernel(nj, c_ref, m_ref, tj_ref, ti_ref, x_ref, w_out_ref,
                 b_out_ref, gamma_ref, beta_ref, out_ref, wsum_ref, cnt_ref):
    j = pl.program_id(1)

    @pl.when(j == 0)
    def _():
        out_ref[...] = jnp.zeros_like(out_ref)
        wsum_ref[...] = jnp.zeros_like(wsum_ref)
        cnt_ref[...] = jnp.zeros_like(cnt_ref)

    inv = 1.0 / (c_ref[...] + _EPS)
    nrm = m_ref[...] * inv
    mask = nrm > _THRESHOLD
    s = jnp.where(mask, nrm, 0.0)
    out_ref[...] += jax.lax.dot_general(
        s, tj_ref[...], (((0,), (0,)), ((), ())),
        preferred_element_type=jnp.float32,
    )
    wsum_ref[...] += jnp.sum(s, axis=0, keepdims=True)
    cnt_ref[...] += jnp.sum(mask.astype(jnp.float32), axis=0, keepdims=True)

    @pl.when(j == nj - 1)
    def _():
        ws = out_ref[...]
        wsum = jnp.transpose(wsum_ref[...])
        has = jnp.transpose(cnt_ref[...]) > 0.0
        agg = ws / (wsum + _EPS)
        agg = jnp.where(has, agg, ti_ref[...])
        o = (
            jax.lax.dot_general(
                agg, w_out_ref[...], (((1,), (1,)), ((), ())),
                preferred_element_type=jnp.float32,
            )
            + b_out_ref[...]
            + x_ref[...]
        )
        mu = jnp.mean(o, axis=1, keepdims=True)
        var = jnp.mean((o - mu) ** 2, axis=1, keepdims=True)
        out_ref[...] = (o - mu) * jax.lax.rsqrt(var + _LN_EPS) * gamma_ref[...] + beta_ref[...]


@functools.partial(jax.jit, static_argnames=())
def kernel(node_features, mobility_matrix, W_in, b_in, W_out, b_out, gamma, beta):
    n, d_in = node_features.shape
    d_out = W_in.shape[0]

    bja = 400
    bi = 2048
    bj = 1000
    nja = n // bja
    ni = pl.cdiv(n, bi)
    nj = n // bj

    b_in2 = b_in.reshape(1, d_out)
    b_out2 = b_out.reshape(1, d_out)
    gamma2 = gamma.reshape(1, d_out)
    beta2 = beta.reshape(1, d_out)

    c, t = pl.pallas_call(
        _prep_kernel,
        grid=(nja,),
        in_specs=[
            pl.BlockSpec((bja, n), lambda j: (j, 0)),
            pl.BlockSpec((bja, d_in), lambda j: (j, 0)),
            pl.BlockSpec((d_out, d_in), lambda j: (0, 0)),
            pl.BlockSpec((1, d_out), lambda j: (0, 0)),
        ],
        out_specs=[
            pl.BlockSpec((1, n), lambda j: (0, 0)),
            pl.BlockSpec((bja, d_out), lambda j: (j, 0)),
        ],
        out_shape=[
            jax.ShapeDtypeStruct((1, n), jnp.float32),
            jax.ShapeDtypeStruct((n, d_out), jnp.float32),
        ],
        compiler_params=pltpu.CompilerParams(
            dimension_semantics=("arbitrary",),
        ),
    )(mobility_matrix, node_features, W_in, b_in2)

    out = pl.pallas_call(
        functools.partial(_main_kernel, nj),
        grid=(ni, nj),
        in_specs=[
            pl.BlockSpec((1, bi), lambda i, j: (0, i)),
            pl.BlockSpec((bj, bi), lambda i, j: (j, i)),
            pl.BlockSpec((bj, d_out), lambda i, j: (j, 0)),
            pl.BlockSpec((bi, d_out), lambda i, j: (i, 0)),
            pl.BlockSpec((bi, d_in), lambda i, j: (i, 0)),
            pl.BlockSpec((d_out, d_out), lambda i, j: (0, 0)),
            pl.BlockSpec((1, d_out), lambda i, j: (0, 0)),
            pl.BlockSpec((1, d_out), lambda i, j: (0, 0)),
            pl.BlockSpec((1, d_out), lambda i, j: (0, 0)),
        ],
        out_specs=pl.BlockSpec((bi, d_out), lambda i, j: (i, 0)),
        out_shape=jax.ShapeDtypeStruct((n, d_out), jnp.float32),
        scratch_shapes=[
            pltpu.VMEM((1, bi), jnp.float32),
            pltpu.VMEM((1, bi), jnp.float32),
        ],
        compiler_params=pltpu.CompilerParams(
            dimension_semantics=("parallel", "arbitrary"),
        ),
    )(c, mobility_matrix, t, t, node_features, W_out, b_out2, gamma2, beta2)

    return out


# speedup vs baseline: 3.7839x; 2.2138x over previous
import functools

import jax
import jax.numpy as jnp
from jax.experimental import pallas as pl
from jax.experimental.pallas import tpu as pltpu

_EPS = 1e-8
_THRESHOLD = 1e-6
_LN_EPS = 1e-5


def _slab_kernel(bi, m_ref, x_full_ref, w_in_ref, b_in_ref, xi_ref,
                 w_out_ref, b_out_ref, gamma_ref, beta_ref, out_ref, t_ref):
    i = pl.program_id(0)

    @pl.when(i == 0)
    def _():
        n = x_full_ref.shape[0]
        t_ref[pl.ds(0, n), :] = (
            jax.lax.dot_general(
                x_full_ref[...], w_in_ref[...], (((1,), (1,)), ((), ())),
                preferred_element_type=jnp.float32,
            )
            + b_in_ref[...]
        )

    m = m_ref[...]
    c = jnp.sum(m, axis=0, keepdims=True)
    s = jnp.where(m > _THRESHOLD * (c + _EPS), m, 0.0)
    wsum = jnp.sum(s, axis=0, keepdims=True)
    t = t_ref[pl.ds(0, m.shape[0]), :]
    ws_t = jax.lax.dot_general(
        t, s, (((0,), (0,)), ((), ())),
        preferred_element_type=jnp.float32,
    )
    inv = 1.0 / (c + _EPS)
    factor = inv / (wsum * inv + _EPS)
    has = wsum > 0.0
    ti_t = jnp.transpose(t_ref[pl.ds(i * bi, bi), :])
    agg_t = jnp.where(has, ws_t * factor, ti_t)
    o_t = (
        jax.lax.dot_general(
            w_out_ref[...], agg_t, (((1,), (0,)), ((), ())),
            preferred_element_type=jnp.float32,
        )
        + jnp.transpose(b_out_ref[...])
        + jnp.transpose(xi_ref[...])
    )
    d = o_t.shape[0]
    mu = jnp.mean(o_t, axis=0, keepdims=True)
    var = jnp.mean((o_t - mu) ** 2, axis=0, keepdims=True)
    n_t = (o_t - mu) * jax.lax.rsqrt(var + _LN_EPS)
    out_ref[...] = jnp.transpose(
        n_t * jnp.transpose(gamma_ref[...]) + jnp.transpose(beta_ref[...])
    )


@jax.jit
def kernel(node_features, mobility_matrix, W_in, b_in, W_out, b_out, gamma, beta):
    n, d_in = node_features.shape
    d_out = W_in.shape[0]

    bi = 512
    ni = pl.cdiv(n, bi)

    b_in2 = b_in.reshape(1, d_out)
    b_out2 = b_out.reshape(1, d_out)
    gamma2 = gamma.reshape(1, d_out)
    beta2 = beta.reshape(1, d_out)

    out = pl.pallas_call(
        functools.partial(_slab_kernel, bi),
        grid=(ni,),
        in_specs=[
            pl.BlockSpec((n, bi), lambda i: (0, i)),
            pl.BlockSpec((n, d_in), lambda i: (0, 0)),
            pl.BlockSpec((d_out, d_in), lambda i: (0, 0)),
            pl.BlockSpec((1, d_out), lambda i: (0, 0)),
            pl.BlockSpec((bi, d_in), lambda i: (i, 0)),
            pl.BlockSpec((d_out, d_out), lambda i: (0, 0)),
            pl.BlockSpec((1, d_out), lambda i: (0, 0)),
            pl.BlockSpec((1, d_out), lambda i: (0, 0)),
            pl.BlockSpec((1, d_out), lambda i: (0, 0)),
        ],
        out_specs=pl.BlockSpec((bi, d_out), lambda i: (i, 0)),
        out_shape=jax.ShapeDtypeStruct((n, d_out), jnp.float32),
        scratch_shapes=[
            pltpu.VMEM((ni * bi, d_out), jnp.float32),
        ],
        compiler_params=pltpu.CompilerParams(
            dimension_semantics=("arbitrary",),
        ),
    )(mobility_matrix, node_features, W_in, b_in2, node_features,
      W_out, b_out2, gamma2, beta2)

    return out
